# baseline (speedup 1.0000x reference)
import jax
import jax.numpy as jnp
from jax.experimental import pallas as pl
from jax.experimental.pallas import tpu as pltpu


_PAR = pltpu.CompilerParams(dimension_semantics=("parallel",))
_BF = jnp.bfloat16
_F32 = jnp.float32


def _dot(a, b):
    return jnp.dot(a, b, preferred_element_type=_F32)


def _row(r, c):
    return pl.BlockSpec((r, c), lambda i: (i, 0))


def _full(a):
    return pl.BlockSpec(a.shape, lambda i: (0, 0))


def _pre_body(f1_ref, f2_ref, ws1_ref, ws2_ref,
              s1_ref, s2_ref, f1b_ref, f2b_ref):
    f1 = f1_ref[...]
    f2 = f2_ref[...]
    s1_ref[...] = _dot(f1.astype(_BF), ws1_ref[...].astype(_BF))
    s2_ref[...] = _dot(f2.astype(_BF), ws2_ref[...].astype(_BF))
    f1b_ref[...] = f1.astype(_BF)
    f2b_ref[...] = f2.astype(_BF)


def _pre(f1, f2, ws1, ws2, block_rows=2000):
    n, d1 = f1.shape
    d2 = f2.shape[1]
    o = ws1.shape[1]
    r = block_rows if n % block_rows == 0 else n
    return pl.pallas_call(
        _pre_body, grid=(n // r,),
        in_specs=[_row(r, d1), _row(r, d2), _full(ws1), _full(ws2)],
        out_specs=[_row(r, o), _row(r, o), _row(r, d1), _row(r, d2)],
        out_shape=[jax.ShapeDtypeStruct((n, o), _F32),
                   jax.ShapeDtypeStruct((n, o), _F32),
                   jax.ShapeDtypeStruct((n, d1), _BF),
                   jax.ShapeDtypeStruct((n, d2), _BF)],
        compiler_params=_PAR,
    )(f1, f2, ws1, ws2)


def _sage_pair_body(asp_ref, aft_ref, f_ref, s_ref, wn_ref,
                    esp_ref, eft_ref):
    f = f_ref[...]
    wn = wn_ref[...].astype(_BF)
    s = s_ref[...]
    nsp = _dot(asp_ref[...].astype(_BF), f)
    esp_ref[...] = jnp.maximum(_dot(nsp.astype(_BF), wn) + s, 0.0)
    nft = _dot(aft_ref[...].astype(_BF), f)
    eft_ref[...] = jnp.maximum(_dot(nft.astype(_BF), wn) + s, 0.0)


def _sage_pair(asp, aft, f_bf16, s, wn, block_rows=80):
    n, k = asp.shape
    d = f_bf16.shape[1]
    o = wn.shape[1]
    r = block_rows if n % block_rows == 0 else n
    return pl.pallas_call(
        _sage_pair_body, grid=(n // r,),
        in_specs=[_row(r, k), _row(r, k), _full(f_bf16), _row(r, o),
                  _full(wn)],
        out_specs=[_row(r, o), _row(r, o)],
        out_shape=[jax.ShapeDtypeStruct((n, o), _F32),
                   jax.ShapeDtypeStruct((n, o), _F32)],
        compiler_params=_PAR,
    )(asp, aft, f_bf16, s, wn)


def _attn_pair(e1, e2, w, u):
    dot = lambda a, b: _dot(a.astype(_BF), b.astype(_BF))
    s1 = dot(jnp.tanh(dot(e1, w)), u)
    s2 = dot(jnp.tanh(dot(e2, w)), u)
    m = jnp.maximum(s1, s2)
    x1 = jnp.exp(s1 - m)
    x2 = jnp.exp(s2 - m)
    den = x1 + x2
    a1 = x1 / den
    a2 = x2 / den
    lat = (e1.astype(_BF).astype(_F32) * a1.astype(_BF).astype(_F32)
           + e2.astype(_BF).astype(_F32) * a2.astype(_BF).astype(_F32))
    return lat, a1, a2


def _attn_body(esp1_ref, eft1_ref, esp2_ref, eft2_ref,
               w1_ref, u1_ref, w2_ref, u2_ref, wc_ref, uc_ref,
               lat1_ref, lat2_ref, comb_ref, ap_ref, cu1_ref, cu2_ref):
    lat1, a10, a11 = _attn_pair(esp1_ref[...], eft1_ref[...],
                                w1_ref[...], u1_ref[...])
    lat2, a20, a21 = _attn_pair(esp2_ref[...], eft2_ref[...],
                                w2_ref[...], u2_ref[...])
    comb, ac0, ac1 = _attn_pair(lat1, lat2, wc_ref[...], uc_ref[...])
    lat1_ref[...] = lat1
    lat2_ref[...] = lat2
    comb_ref[...] = comb
    r = lat1.shape[0]
    ap_ref[...] = jnp.concatenate(
        [a10, a11, a20, a21, ac0, ac1,
         jnp.zeros((r, 122), _F32)], axis=1)
    cu1_ref[...] = jnp.concatenate([comb, lat2], axis=1).astype(_BF)
    cu2_ref[...] = jnp.concatenate([comb, lat1], axis=1).astype(_BF)


def _attention(esp1, eft1, esp2, eft2, w1, u1, w2, u2, wc, uc,
               block_rows=2000):
    n, o = esp1.shape
    r = block_rows if n % block_rows == 0 else n
    return pl.pallas_call(
        _attn_body, grid=(n // r,),
        in_specs=[_row(r, o), _row(r, o), _row(r, o), _row(r, o),
                  _full(w1), _full(u1), _full(w2), _full(u2),
                  _full(wc), _full(uc)],
        out_specs=[_row(r, o), _row(r, o), _row(r, o), _row(r, 128),
                   _row(r, 2 * o), _row(r, 2 * o)],
        out_shape=[jax.ShapeDtypeStruct((n, o), _F32),
                   jax.ShapeDtypeStruct((n, o), _F32),
                   jax.ShapeDtypeStruct((n, o), _F32),
                   jax.ShapeDtypeStruct((n, 128), _F32),
                   jax.ShapeDtypeStruct((n, 2 * o), _BF),
                   jax.ShapeDtypeStruct((n, 2 * o), _BF)],
        compiler_params=_PAR,
    )(esp1, eft1, esp2, eft2, w1, u1, w2, u2, wc, uc)


def _dual_spmm_body(a1_ref, a2_ref, h1_ref, h2_ref, v1_ref, v2_ref):
    v1_ref[...] = _dot(a1_ref[...].astype(_BF), h1_ref[...])
    v2_ref[...] = _dot(a2_ref[...].astype(_BF), h2_ref[...])


def _dual_spmm(a1, a2, h1_bf16, h2_bf16, block_rows=80):
    n, k = a1.shape
    o1 = h1_bf16.shape[1]
    o2 = h2_bf16.shape[1]
    r = block_rows if n % block_rows == 0 else n
    return pl.pallas_call(
        _dual_spmm_body, grid=(n // r,),
        in_specs=[_row(r, k), _row(r, k), _full(h1_bf16), _full(h2_bf16)],
        out_specs=[_row(r, o1), _row(r, o2)],
        out_shape=[jax.ShapeDtypeStruct((n, o1), _F32),
                   jax.ShapeDtypeStruct((n, o2), _F32)],
        compiler_params=_PAR,
    )(a1, a2, h1_bf16, h2_bf16)


def _epi_body(comb_ref, lat1_ref, lat2_ref, v1_ref, v2_ref,
              wsd1_ref, wnd1_ref, wsd2_ref, wnd2_ref,
              ws1_ref, wn1_ref, ws2_ref, wn2_ref,
              rec1_ref, rec2_ref, sa1_ref, ha1_ref, sa2_ref, ha2_ref):
    comb = comb_ref[...]
    o = comb.shape[1]
    v1 = v1_ref[...]
    v2 = v2_ref[...]
    rec1_ref[...] = jnp.maximum(
        _dot(comb, wsd1_ref[...]) + _dot(v1[:, :o], wnd1_ref[...]), 0.0)
    a2in = jnp.maximum(
        _dot(lat2_ref[...], wsd1_ref[...]) + _dot(v1[:, o:], wnd1_ref[...]),
        0.0)
    rec2_ref[...] = jnp.maximum(
        _dot(comb, wsd2_ref[...]) + _dot(v2[:, :o], wnd2_ref[...]), 0.0)
    a1in = jnp.maximum(
        _dot(lat1_ref[...], wsd2_ref[...]) + _dot(v2[:, o:], wnd2_ref[...]),
        0.0)
    sa1_ref[...] = _dot(a1in, ws2_ref[...])
    ha1_ref[...] = _dot(a1in, wn2_ref[...]).astype(_BF)
    sa2_ref[...] = _dot(a2in, ws1_ref[...])
    ha2_ref[...] = _dot(a2in, wn1_ref[...]).astype(_BF)


def _epilogue(comb, lat1, lat2, v1, v2,
              Wsd1, Wnd1, Wsd2, Wnd2, Ws1, Wn1, Ws2, Wn2,
              block_rows=2000):
    n, o = comb.shape
    d1 = Wsd1.shape[1]
    d2 = Wsd2.shape[1]
    r = block_rows if n % block_rows == 0 else n
    return pl.pallas_call(
        _epi_body, grid=(n // r,),
        in_specs=[_row(r, o), _row(r, o), _row(r, o),
                  _row(r, 2 * o), _row(r, 2 * o),
                  _full(Wsd1), _full(Wnd1), _full(Wsd2), _full(Wnd2),
                  _full(Ws1), _full(Wn1), _full(Ws2), _full(Wn2)],
        out_specs=[_row(r, d1), _row(r, d2), _row(r, o), _row(r, o),
                   _row(r, o), _row(r, o)],
        out_shape=[jax.ShapeDtypeStruct((n, d1), _F32),
                   jax.ShapeDtypeStruct((n, d2), _F32),
                   jax.ShapeDtypeStruct((n, o), _F32),
                   jax.ShapeDtypeStruct((n, o), _BF),
                   jax.ShapeDtypeStruct((n, o), _F32),
                   jax.ShapeDtypeStruct((n, o), _BF)],
        compiler_params=_PAR,
    )(comb, lat1, lat2, v1, v2,
      Wsd1, Wnd1, Wsd2, Wnd2, Ws1, Wn1, Ws2, Wn2)


def _dual_sage_body(a2_ref, a1_ref, h1_ref, h2_ref, s1_ref, s2_ref,
                    o1_ref, o2_ref):
    o1_ref[...] = jnp.maximum(
        _dot(a2_ref[...].astype(_BF), h1_ref[...]) + s1_ref[...], 0.0)
    o2_ref[...] = jnp.maximum(
        _dot(a1_ref[...].astype(_BF), h2_ref[...]) + s2_ref[...], 0.0)


def _dual_sage(a2, a1, h1_bf16, h2_bf16, s1, s2, block_rows=80):
    n, k = a2.shape
    o = h1_bf16.shape[1]
    r = block_rows if n % block_rows == 0 else n
    return pl.pallas_call(
        _dual_sage_body, grid=(n // r,),
        in_specs=[_row(r, k), _row(r, k), _full(h1_bf16), _full(h2_bf16),
                  _row(r, o), _row(r, o)],
        out_specs=[_row(r, o), _row(r, o)],
        out_shape=[jax.ShapeDtypeStruct((n, o), _F32),
                   jax.ShapeDtypeStruct((n, o), _F32)],
        compiler_params=_PAR,
    )(a2, a1, h1_bf16, h2_bf16, s1, s2)


def kernel(features_omics1, features_omics2, adj_spatial_omics1,
           adj_feature_omics1, adj_spatial_omics2, adj_feature_omics2,
           Ws1, Wn1, Wsd1, Wnd1, Ws2, Wn2, Wsd2, Wnd2,
           w1, u1, w2, u2, wc, uc):
    s1, s2, f1b, f2b = _pre(features_omics1, features_omics2, Ws1, Ws2)

    e_sp1, e_ft1 = _sage_pair(adj_spatial_omics1, adj_feature_omics1,
                              f1b, s1, Wn1)
    e_sp2, e_ft2 = _sage_pair(adj_spatial_omics2, adj_feature_omics2,
                              f2b, s2, Wn2)

    lat1, lat2, comb, ap, cu1, cu2 = _attention(
        e_sp1, e_ft1, e_sp2, e_ft2, w1, u1, w2, u2, wc, uc)
    alpha1 = ap[:, 0:2]
    alpha2 = ap[:, 2:4]
    alpha12 = ap[:, 4:6]

    v1, v2 = _dual_spmm(adj_spatial_omics1, adj_spatial_omics2, cu1, cu2)

    rec1, rec2, sa1, ha1, sa2, ha2 = _epilogue(
        comb, lat1, lat2, v1, v2,
        Wsd1, Wnd1, Wsd2, Wnd2, Ws1, Wn1, Ws2, Wn2)

    across1, across2 = _dual_sage(adj_spatial_omics2, adj_spatial_omics1,
                                  ha1, ha2, sa1, sa2)

    return (lat1, lat2, comb, rec1, rec2, across1, across2,
            alpha1, alpha2, alpha12)


# speedup vs baseline: 1.1709x; 1.1709x over previous
import jax
import jax.numpy as jnp
from jax.experimental import pallas as pl
from jax.experimental.pallas import tpu as pltpu


_PAR = pltpu.CompilerParams(dimension_semantics=("parallel",))
_BF = jnp.bfloat16
_F32 = jnp.float32


def _dot(a, b):
    return jnp.dot(a, b, preferred_element_type=_F32)


def _row(r, c):
    return pl.BlockSpec((r, c), lambda i: (i, 0))


def _full(a):
    return pl.BlockSpec(a.shape, lambda i: (0, 0))


def _mm_body(x_ref, w_ref, o_ref):
    o_ref[...] = _dot(x_ref[...], w_ref[...])


def _mm(x, w, block_rows=2000):
    n, d = x.shape
    o = w.shape[1]
    r = block_rows if n % block_rows == 0 else n
    return pl.pallas_call(
        _mm_body, grid=(n // r,),
        in_specs=[_row(r, d), _full(w)],
        out_specs=_row(r, o),
        out_shape=jax.ShapeDtypeStruct((n, o), _F32),
        compiler_params=_PAR,
    )(x, w)


def _sage_ref_body(a_ref, f_ref, s_ref, wn_ref, o_ref):
    neigh = _dot(a_ref[...], f_ref[...])
    o_ref[...] = jnp.maximum(_dot(neigh, wn_ref[...]) + s_ref[...], 0.0)


def _sage_ref(adj, feat, s, wn, block_rows=200):
    n, k = adj.shape
    d = feat.shape[1]
    o = wn.shape[1]
    r = block_rows if n % block_rows == 0 else n
    return pl.pallas_call(
        _sage_ref_body, grid=(n // r,),
        in_specs=[_row(r, k), _full(feat), _row(r, o), _full(wn)],
        out_specs=_row(r, o),
        out_shape=jax.ShapeDtypeStruct((n, o), _F32),
        compiler_params=_PAR,
    )(adj, feat, s, wn)


def _attn_pair(e1, e2, w, u):
    s1 = _dot(jnp.tanh(_dot(e1, w)), u)
    s2 = _dot(jnp.tanh(_dot(e2, w)), u)
    m = jnp.maximum(s1, s2)
    x1 = jnp.exp(s1 - m)
    x2 = jnp.exp(s2 - m)
    den = x1 + x2
    a1 = x1 / den
    a2 = x2 / den
    lat = (e1.astype(_BF).astype(_F32) * a1.astype(_BF).astype(_F32)
           + e2.astype(_BF).astype(_F32) * a2.astype(_BF).astype(_F32))
    return lat, a1, a2


def _attn_body(esp1_ref, eft1_ref, esp2_ref, eft2_ref,
               w1_ref, u1_ref, w2_ref, u2_ref, wc_ref, uc_ref,
               lat1_ref, lat2_ref, comb_ref, ap_ref, cu1_ref, cu2_ref):
    lat1, a10, a11 = _attn_pair(esp1_ref[...], eft1_ref[...],
                                w1_ref[...], u1_ref[...])
    lat2, a20, a21 = _attn_pair(esp2_ref[...], eft2_ref[...],
                                w2_ref[...], u2_ref[...])
    comb, ac0, ac1 = _attn_pair(lat1, lat2, wc_ref[...], uc_ref[...])
    lat1_ref[...] = lat1
    lat2_ref[...] = lat2
    comb_ref[...] = comb
    r = lat1.shape[0]
    ap_ref[...] = jnp.concatenate(
        [a10, a11, a20, a21, ac0, ac1,
         jnp.zeros((r, 122), _F32)], axis=1)
    cu1_ref[...] = jnp.concatenate([comb, lat2], axis=1)
    cu2_ref[...] = jnp.concatenate([comb, lat1], axis=1)


def _attention(esp1, eft1, esp2, eft2, w1, u1, w2, u2, wc, uc,
               block_rows=2000):
    n, o = esp1.shape
    r = block_rows if n % block_rows == 0 else n
    return pl.pallas_call(
        _attn_body, grid=(n // r,),
        in_specs=[_row(r, o), _row(r, o), _row(r, o), _row(r, o),
                  _full(w1), _full(u1), _full(w2), _full(u2),
                  _full(wc), _full(uc)],
        out_specs=[_row(r, o), _row(r, o), _row(r, o), _row(r, 128),
                   _row(r, 2 * o), _row(r, 2 * o)],
        out_shape=[jax.ShapeDtypeStruct((n, o), _F32),
                   jax.ShapeDtypeStruct((n, o), _F32),
                   jax.ShapeDtypeStruct((n, o), _F32),
                   jax.ShapeDtypeStruct((n, 128), _F32),
                   jax.ShapeDtypeStruct((n, 2 * o), _F32),
                   jax.ShapeDtypeStruct((n, 2 * o), _F32)],
        compiler_params=_PAR,
    )(esp1, eft1, esp2, eft2, w1, u1, w2, u2, wc, uc)


def _spmm_body(a_ref, h_ref, o_ref):
    o_ref[...] = _dot(a_ref[...], h_ref[...])


def _spmm(adj, h, block_rows=200):
    n, k = adj.shape
    o = h.shape[1]
    r = block_rows if n % block_rows == 0 else n
    return pl.pallas_call(
        _spmm_body, grid=(n // r,),
        in_specs=[_row(r, k), _full(h)],
        out_specs=_row(r, o),
        out_shape=jax.ShapeDtypeStruct((n, o), _F32),
        compiler_params=_PAR,
    )(adj, h)


def _sage_fast_body(a_ref, h_ref, s_ref, o_ref):
    o_ref[...] = jnp.maximum(
        _dot(a_ref[...], h_ref[...]) + s_ref[...], 0.0)


def _sage_fast(adj, h, s, block_rows=400):
    n, k = adj.shape
    o = h.shape[1]
    r = block_rows if n % block_rows == 0 else n
    return pl.pallas_call(
        _sage_fast_body, grid=(n // r,),
        in_specs=[_row(r, k), _full(h), _row(r, o)],
        out_specs=_row(r, o),
        out_shape=jax.ShapeDtypeStruct((n, o), _F32),
        compiler_params=_PAR,
    )(adj, h, s)


def _epi_body(comb_ref, lat1_ref, lat2_ref, v1_ref, v2_ref,
              wsd1_ref, wnd1_ref, wsd2_ref, wnd2_ref,
              ws1_ref, wn1_ref, ws2_ref, wn2_ref,
              rec1_ref, rec2_ref, sa1_ref, ha1_ref, sa2_ref, ha2_ref):
    comb = comb_ref[...]
    o = comb.shape[1]
    v1 = v1_ref[...]
    v2 = v2_ref[...]
    rec1_ref[...] = jnp.maximum(
        _dot(comb, wsd1_ref[...]) + _dot(v1[:, :o], wnd1_ref[...]), 0.0)
    a2in = jnp.maximum(
        _dot(lat2_ref[...], wsd1_ref[...]) + _dot(v1[:, o:], wnd1_ref[...]),
        0.0)
    rec2_ref[...] = jnp.maximum(
        _dot(comb, wsd2_ref[...]) + _dot(v2[:, :o], wnd2_ref[...]), 0.0)
    a1in = jnp.maximum(
        _dot(lat1_ref[...], wsd2_ref[...]) + _dot(v2[:, o:], wnd2_ref[...]),
        0.0)
    sa1_ref[...] = _dot(a1in, ws2_ref[...])
    ha1_ref[...] = _dot(a1in, wn2_ref[...])
    sa2_ref[...] = _dot(a2in, ws1_ref[...])
    ha2_ref[...] = _dot(a2in, wn1_ref[...])


def _epilogue(comb, lat1, lat2, v1, v2,
              Wsd1, Wnd1, Wsd2, Wnd2, Ws1, Wn1, Ws2, Wn2,
              block_rows=2000):
    n, o = comb.shape
    d1 = Wsd1.shape[1]
    d2 = Wsd2.shape[1]
    r = block_rows if n % block_rows == 0 else n
    return pl.pallas_call(
        _epi_body, grid=(n // r,),
        in_specs=[_row(r, o), _row(r, o), _row(r, o),
                  _row(r, 2 * o), _row(r, 2 * o),
                  _full(Wsd1), _full(Wnd1), _full(Wsd2), _full(Wnd2),
                  _full(Ws1), _full(Wn1), _full(Ws2), _full(Wn2)],
        out_specs=[_row(r, d1), _row(r, d2), _row(r, o), _row(r, o),
                   _row(r, o), _row(r, o)],
        out_shape=[jax.ShapeDtypeStruct((n, d1), _F32),
                   jax.ShapeDtypeStruct((n, d2), _F32),
                   jax.ShapeDtypeStruct((n, o), _F32),
                   jax.ShapeDtypeStruct((n, o), _F32),
                   jax.ShapeDtypeStruct((n, o), _F32),
                   jax.ShapeDtypeStruct((n, o), _F32)],
        compiler_params=_PAR,
    )(comb, lat1, lat2, v1, v2,
      Wsd1, Wnd1, Wsd2, Wnd2, Ws1, Wn1, Ws2, Wn2)


def kernel(features_omics1, features_omics2, adj_spatial_omics1,
           adj_feature_omics1, adj_spatial_omics2, adj_feature_omics2,
           Ws1, Wn1, Wsd1, Wnd1, Ws2, Wn2, Wsd2, Wnd2,
           w1, u1, w2, u2, wc, uc):
    s1 = _mm(features_omics1, Ws1)
    s2 = _mm(features_omics2, Ws2)

    e_sp1 = _sage_ref(adj_spatial_omics1, features_omics1, s1, Wn1)
    e_ft1 = _sage_ref(adj_feature_omics1, features_omics1, s1, Wn1)
    e_sp2 = _sage_ref(adj_spatial_omics2, features_omics2, s2, Wn2)
    e_ft2 = _sage_ref(adj_feature_omics2, features_omics2, s2, Wn2)

    lat1, lat2, comb, ap, cu1, cu2 = _attention(
        e_sp1, e_ft1, e_sp2, e_ft2, w1, u1, w2, u2, wc, uc)
    alpha1 = ap[:, 0:2]
    alpha2 = ap[:, 2:4]
    alpha12 = ap[:, 4:6]

    v1 = _spmm(adj_spatial_omics1, cu1)
    v2 = _spmm(adj_spatial_omics2, cu2)

    rec1, rec2, sa1, ha1, sa2, ha2 = _epilogue(
        comb, lat1, lat2, v1, v2,
        Wsd1, Wnd1, Wsd2, Wnd2, Ws1, Wn1, Ws2, Wn2)

    across1 = _sage_fast(adj_spatial_omics2, ha1, sa1)
    across2 = _sage_fast(adj_spatial_omics1, ha2, sa2)

    return (lat1, lat2, comb, rec1, rec2, across1, across2,
            alpha1, alpha2, alpha12)


# speedup vs baseline: 1.1848x; 1.0119x over previous
import functools

import jax
import jax.numpy as jnp
from jax.experimental import pallas as pl
from jax.experimental.pallas import tpu as pltpu


_PAR = pltpu.CompilerParams(dimension_semantics=("parallel",))
_BF = jnp.bfloat16
_F32 = jnp.float32


def _dot(a, b):
    return jnp.dot(a, b, preferred_element_type=_F32)


def _row(r, c):
    return pl.BlockSpec((r, c), lambda i: (i, 0))


def _full(a):
    return pl.BlockSpec(a.shape, lambda i: (0, 0))


def _sage_pair_body(asp_ref, aft_ref, f_ref, ws_ref, wn_ref,
                    esp_ref, eft_ref, *, r):
    i = pl.program_id(0)
    f = f_ref[...]
    wn = wn_ref[...]
    s = _dot(f_ref[pl.ds(i * r, r), :], ws_ref[...])
    nsp = _dot(asp_ref[...], f)
    esp_ref[...] = jnp.maximum(_dot(nsp, wn) + s, 0.0)
    nft = _dot(aft_ref[...], f)
    eft_ref[...] = jnp.maximum(_dot(nft, wn) + s, 0.0)


def _sage_pair(asp, aft, feat, ws, wn, block_rows=200):
    n, k = asp.shape
    d = feat.shape[1]
    o = wn.shape[1]
    r = block_rows if n % block_rows == 0 else n
    return pl.pallas_call(
        functools.partial(_sage_pair_body, r=r), grid=(n // r,),
        in_specs=[_row(r, k), _row(r, k), _full(feat), _full(ws),
                  _full(wn)],
        out_specs=[_row(r, o), _row(r, o)],
        out_shape=[jax.ShapeDtypeStruct((n, o), _F32),
                   jax.ShapeDtypeStruct((n, o), _F32)],
        compiler_params=_PAR,
    )(asp, aft, feat, ws, wn)


def _attn_pair(e1, e2, w, u):
    s1 = _dot(jnp.tanh(_dot(e1, w)), u)
    s2 = _dot(jnp.tanh(_dot(e2, w)), u)
    m = jnp.maximum(s1, s2)
    x1 = jnp.exp(s1 - m)
    x2 = jnp.exp(s2 - m)
    den = x1 + x2
    a1 = x1 / den
    a2 = x2 / den
    lat = (e1.astype(_BF).astype(_F32) * a1.astype(_BF).astype(_F32)
           + e2.astype(_BF).astype(_F32) * a2.astype(_BF).astype(_F32))
    return lat, a1, a2


def _attn_body(esp1_ref, eft1_ref, esp2_ref, eft2_ref,
               w1_ref, u1_ref, w2_ref, u2_ref, wc_ref, uc_ref,
               lat1_ref, lat2_ref, comb_ref, ap_ref, cu1_ref, cu2_ref):
    lat1, a10, a11 = _attn_pair(esp1_ref[...], eft1_ref[...],
                                w1_ref[...], u1_ref[...])
    lat2, a20, a21 = _attn_pair(esp2_ref[...], eft2_ref[...],
                                w2_ref[...], u2_ref[...])
    comb, ac0, ac1 = _attn_pair(lat1, lat2, wc_ref[...], uc_ref[...])
    lat1_ref[...] = lat1
    lat2_ref[...] = lat2
    comb_ref[...] = comb
    r = lat1.shape[0]
    ap_ref[...] = jnp.concatenate(
        [a10, a11, a20, a21, ac0, ac1,
         jnp.zeros((r, 122), _F32)], axis=1)
    cu1_ref[...] = jnp.concatenate([comb, lat2], axis=1)
    cu2_ref[...] = jnp.concatenate([comb, lat1], axis=1)


def _attention(esp1, eft1, esp2, eft2, w1, u1, w2, u2, wc, uc,
               block_rows=2000):
    n, o = esp1.shape
    r = block_rows if n % block_rows == 0 else n
    return pl.pallas_call(
        _attn_body, grid=(n // r,),
        in_specs=[_row(r, o), _row(r, o), _row(r, o), _row(r, o),
                  _full(w1), _full(u1), _full(w2), _full(u2),
                  _full(wc), _full(uc)],
        out_specs=[_row(r, o), _row(r, o), _row(r, o), _row(r, 128),
                   _row(r, 2 * o), _row(r, 2 * o)],
        out_shape=[jax.ShapeDtypeStruct((n, o), _F32),
                   jax.ShapeDtypeStruct((n, o), _F32),
                   jax.ShapeDtypeStruct((n, o), _F32),
                   jax.ShapeDtypeStruct((n, 128), _F32),
                   jax.ShapeDtypeStruct((n, 2 * o), _F32),
                   jax.ShapeDtypeStruct((n, 2 * o), _F32)],
        compiler_params=_PAR,
    )(esp1, eft1, esp2, eft2, w1, u1, w2, u2, wc, uc)


def _spmm_epi_body(a_ref, cu_ref, comb_ref, latb_ref,
                   wsd_ref, wnd_ref, ws_ref, wn_ref,
                   rec_ref, sa_ref, ha_ref):
    o = comb_ref.shape[1]
    v = _dot(a_ref[...], cu_ref[...])
    wsd = wsd_ref[...]
    wnd = wnd_ref[...]
    rec_ref[...] = jnp.maximum(
        _dot(comb_ref[...], wsd) + _dot(v[:, :o], wnd), 0.0)
    ain = jnp.maximum(
        _dot(latb_ref[...], wsd) + _dot(v[:, o:], wnd), 0.0)
    sa_ref[...] = _dot(ain, ws_ref[...])
    ha_ref[...] = _dot(ain, wn_ref[...])


def _spmm_epi(adj, cu, comb, latb, wsd, wnd, ws, wn, block_rows=200):
    n, k = adj.shape
    o = comb.shape[1]
    dd = wsd.shape[1]
    oo = ws.shape[1]
    r = block_rows if n % block_rows == 0 else n
    return pl.pallas_call(
        _spmm_epi_body, grid=(n // r,),
        in_specs=[_row(r, k), _full(cu), _row(r, o), _row(r, o),
                  _full(wsd), _full(wnd), _full(ws), _full(wn)],
        out_specs=[_row(r, dd), _row(r, oo), _row(r, oo)],
        out_shape=[jax.ShapeDtypeStruct((n, dd), _F32),
                   jax.ShapeDtypeStruct((n, oo), _F32),
                   jax.ShapeDtypeStruct((n, oo), _F32)],
        compiler_params=_PAR,
    )(adj, cu, comb, latb, wsd, wnd, ws, wn)


def _dual_sage_body(a2_ref, a1_ref, h1_ref, h2_ref, s1_ref, s2_ref,
                    o1_ref, o2_ref):
    o1_ref[...] = jnp.maximum(
        _dot(a2_ref[...], h1_ref[...]) + s1_ref[...], 0.0)
    o2_ref[...] = jnp.maximum(
        _dot(a1_ref[...], h2_ref[...]) + s2_ref[...], 0.0)


def _dual_sage(a2, a1, h1, h2, s1, s2, block_rows=200):
    n, k = a2.shape
    o = h1.shape[1]
    r = block_rows if n % block_rows == 0 else n
    return pl.pallas_call(
        _dual_sage_body, grid=(n // r,),
        in_specs=[_row(r, k), _row(r, k), _full(h1), _full(h2),
                  _row(r, o), _row(r, o)],
        out_specs=[_row(r, o), _row(r, o)],
        out_shape=[jax.ShapeDtypeStruct((n, o), _F32),
                   jax.ShapeDtypeStruct((n, o), _F32)],
        compiler_params=_PAR,
    )(a2, a1, h1, h2, s1, s2)


def kernel(features_omics1, features_omics2, adj_spatial_omics1,
           adj_feature_omics1, adj_spatial_omics2, adj_feature_omics2,
           Ws1, Wn1, Wsd1, Wnd1, Ws2, Wn2, Wsd2, Wnd2,
           w1, u1, w2, u2, wc, uc):
    e_sp1, e_ft1 = _sage_pair(adj_spatial_omics1, adj_feature_omics1,
                              features_omics1, Ws1, Wn1)
    e_sp2, e_ft2 = _sage_pair(adj_spatial_omics2, adj_feature_omics2,
                              features_omics2, Ws2, Wn2)

    lat1, lat2, comb, ap, cu1, cu2 = _attention(
        e_sp1, e_ft1, e_sp2, e_ft2, w1, u1, w2, u2, wc, uc)
    alpha1 = ap[:, 0:2]
    alpha2 = ap[:, 2:4]
    alpha12 = ap[:, 4:6]

    rec1, sa2, ha2 = _spmm_epi(adj_spatial_omics1, cu1, comb, lat2,
                               Wsd1, Wnd1, Ws1, Wn1)
    rec2, sa1, ha1 = _spmm_epi(adj_spatial_omics2, cu2, comb, lat1,
                               Wsd2, Wnd2, Ws2, Wn2)

    across1, across2 = _dual_sage(adj_spatial_omics2, adj_spatial_omics1,
                                  ha1, ha2, sa1, sa2)

    return (lat1, lat2, comb, rec1, rec2, across1, across2,
            alpha1, alpha2, alpha12)


# speedup vs baseline: 1.2195x; 1.0293x over previous
import functools

import jax
import jax.numpy as jnp
from jax.experimental import pallas as pl
from jax.experimental.pallas import tpu as pltpu


_PAR = pltpu.CompilerParams(dimension_semantics=("parallel",))
_BF = jnp.bfloat16
_F32 = jnp.float32


def _dot(a, b):
    return jnp.dot(a, b, preferred_element_type=_F32)


def _row(r, c):
    return pl.BlockSpec((r, c), lambda i: (i, 0))


def _full(a):
    return pl.BlockSpec(a.shape, lambda i: (0, 0))


def _sage_pair_body(asp_ref, aft_ref, f_ref, ws_ref, wn_ref,
                    esp_ref, eft_ref, absp_ref, *, r):
    i = pl.program_id(0)
    f = f_ref[...]
    wn = wn_ref[...]
    s = _dot(f_ref[pl.ds(i * r, r), :], ws_ref[...])
    asp = asp_ref[...]
    absp_ref[...] = asp.astype(_BF)
    nsp = _dot(asp, f)
    esp_ref[...] = jnp.maximum(_dot(nsp, wn) + s, 0.0)
    nft = _dot(aft_ref[...], f)
    eft_ref[...] = jnp.maximum(_dot(nft, wn) + s, 0.0)


def _sage_pair(asp, aft, feat, ws, wn, block_rows=200):
    n, k = asp.shape
    d = feat.shape[1]
    o = wn.shape[1]
    r = block_rows if n % block_rows == 0 else n
    return pl.pallas_call(
        functools.partial(_sage_pair_body, r=r), grid=(n // r,),
        in_specs=[_row(r, k), _row(r, k), _full(feat), _full(ws),
                  _full(wn)],
        out_specs=[_row(r, o), _row(r, o), _row(r, k)],
        out_shape=[jax.ShapeDtypeStruct((n, o), _F32),
                   jax.ShapeDtypeStruct((n, o), _F32),
                   jax.ShapeDtypeStruct((n, k), _BF)],
        compiler_params=_PAR,
    )(asp, aft, feat, ws, wn)


def _attn_pair(e1, e2, w, u):
    s1 = _dot(jnp.tanh(_dot(e1, w)), u)
    s2 = _dot(jnp.tanh(_dot(e2, w)), u)
    m = jnp.maximum(s1, s2)
    x1 = jnp.exp(s1 - m)
    x2 = jnp.exp(s2 - m)
    den = x1 + x2
    a1 = x1 / den
    a2 = x2 / den
    lat = (e1.astype(_BF).astype(_F32) * a1.astype(_BF).astype(_F32)
           + e2.astype(_BF).astype(_F32) * a2.astype(_BF).astype(_F32))
    return lat, a1, a2


def _attn_body(esp1_ref, eft1_ref, esp2_ref, eft2_ref,
               w1_ref, u1_ref, w2_ref, u2_ref, wc_ref, uc_ref,
               lat1_ref, lat2_ref, comb_ref, ap_ref, cu1_ref, cu2_ref):
    lat1, a10, a11 = _attn_pair(esp1_ref[...], eft1_ref[...],
                                w1_ref[...], u1_ref[...])
    lat2, a20, a21 = _attn_pair(esp2_ref[...], eft2_ref[...],
                                w2_ref[...], u2_ref[...])
    comb, ac0, ac1 = _attn_pair(lat1, lat2, wc_ref[...], uc_ref[...])
    lat1_ref[...] = lat1
    lat2_ref[...] = lat2
    comb_ref[...] = comb
    r = lat1.shape[0]
    ap_ref[...] = jnp.concatenate(
        [a10, a11, a20, a21, ac0, ac1,
         jnp.zeros((r, 122), _F32)], axis=1)
    cu1_ref[...] = jnp.concatenate([comb, lat2], axis=1).astype(_BF)
    cu2_ref[...] = jnp.concatenate([comb, lat1], axis=1).astype(_BF)


def _attention(esp1, eft1, esp2, eft2, w1, u1, w2, u2, wc, uc,
               block_rows=2000):
    n, o = esp1.shape
    r = block_rows if n % block_rows == 0 else n
    return pl.pallas_call(
        _attn_body, grid=(n // r,),
        in_specs=[_row(r, o), _row(r, o), _row(r, o), _row(r, o),
                  _full(w1), _full(u1), _full(w2), _full(u2),
                  _full(wc), _full(uc)],
        out_specs=[_row(r, o), _row(r, o), _row(r, o), _row(r, 128),
                   _row(r, 2 * o), _row(r, 2 * o)],
        out_shape=[jax.ShapeDtypeStruct((n, o), _F32),
                   jax.ShapeDtypeStruct((n, o), _F32),
                   jax.ShapeDtypeStruct((n, o), _F32),
                   jax.ShapeDtypeStruct((n, 128), _F32),
                   jax.ShapeDtypeStruct((n, 2 * o), _BF),
                   jax.ShapeDtypeStruct((n, 2 * o), _BF)],
        compiler_params=_PAR,
    )(esp1, eft1, esp2, eft2, w1, u1, w2, u2, wc, uc)


def _spmm_epi_body(a_ref, cu_ref, comb_ref, latb_ref,
                   wsd_ref, wnd_ref, ws_ref, wn_ref,
                   rec_ref, sa_ref, ha_ref):
    o = comb_ref.shape[1]
    v = _dot(a_ref[...], cu_ref[...])
    wsd = wsd_ref[...]
    wnd = wnd_ref[...]
    rec_ref[...] = jnp.maximum(
        _dot(comb_ref[...], wsd) + _dot(v[:, :o], wnd), 0.0)
    ain = jnp.maximum(
        _dot(latb_ref[...], wsd) + _dot(v[:, o:], wnd), 0.0)
    sa_ref[...] = _dot(ain, ws_ref[...])
    ha_ref[...] = _dot(ain, wn_ref[...]).astype(_BF)


def _spmm_epi(adj, cu, comb, latb, wsd, wnd, ws, wn, block_rows=200):
    n, k = adj.shape
    o = comb.shape[1]
    dd = wsd.shape[1]
    oo = ws.shape[1]
    r = block_rows if n % block_rows == 0 else n
    return pl.pallas_call(
        _spmm_epi_body, grid=(n // r,),
        in_specs=[_row(r, k), _full(cu), _row(r, o), _row(r, o),
                  _full(wsd), _full(wnd), _full(ws), _full(wn)],
        out_specs=[_row(r, dd), _row(r, oo), _row(r, oo)],
        out_shape=[jax.ShapeDtypeStruct((n, dd), _F32),
                   jax.ShapeDtypeStruct((n, oo), _F32),
                   jax.ShapeDtypeStruct((n, oo), _BF)],
        compiler_params=_PAR,
    )(adj, cu, comb, latb, wsd, wnd, ws, wn)


def _dual_sage_body(a2_ref, a1_ref, h1_ref, h2_ref, s1_ref, s2_ref,
                    o1_ref, o2_ref):
    o1_ref[...] = jnp.maximum(
        _dot(a2_ref[...], h1_ref[...]) + s1_ref[...], 0.0)
    o2_ref[...] = jnp.maximum(
        _dot(a1_ref[...], h2_ref[...]) + s2_ref[...], 0.0)


def _dual_sage(a2, a1, h1, h2, s1, s2, block_rows=200):
    n, k = a2.shape
    o = h1.shape[1]
    r = block_rows if n % block_rows == 0 else n
    return pl.pallas_call(
        _dual_sage_body, grid=(n // r,),
        in_specs=[_row(r, k), _row(r, k), _full(h1), _full(h2),
                  _row(r, o), _row(r, o)],
        out_specs=[_row(r, o), _row(r, o)],
        out_shape=[jax.ShapeDtypeStruct((n, o), _F32),
                   jax.ShapeDtypeStruct((n, o), _F32)],
        compiler_params=_PAR,
    )(a2, a1, h1, h2, s1, s2)


def kernel(features_omics1, features_omics2, adj_spatial_omics1,
           adj_feature_omics1, adj_spatial_omics2, adj_feature_omics2,
           Ws1, Wn1, Wsd1, Wnd1, Ws2, Wn2, Wsd2, Wnd2,
           w1, u1, w2, u2, wc, uc):
    e_sp1, e_ft1, asp1_bf = _sage_pair(
        adj_spatial_omics1, adj_feature_omics1, features_omics1, Ws1, Wn1)
    e_sp2, e_ft2, asp2_bf = _sage_pair(
        adj_spatial_omics2, adj_feature_omics2, features_omics2, Ws2, Wn2)

    lat1, lat2, comb, ap, cu1, cu2 = _attention(
        e_sp1, e_ft1, e_sp2, e_ft2, w1, u1, w2, u2, wc, uc)
    alpha1 = ap[:, 0:2]
    alpha2 = ap[:, 2:4]
    alpha12 = ap[:, 4:6]

    rec1, sa2, ha2 = _spmm_epi(asp1_bf, cu1, comb, lat2,
                               Wsd1, Wnd1, Ws1, Wn1)
    rec2, sa1, ha1 = _spmm_epi(asp2_bf, cu2, comb, lat1,
                               Wsd2, Wnd2, Ws2, Wn2)

    across1, across2 = _dual_sage(asp2_bf, asp1_bf,
                                  ha1, ha2, sa1, sa2)

    return (lat1, lat2, comb, rec1, rec2, across1, across2,
            alpha1, alpha2, alpha12)


# speedup vs baseline: 1.2938x; 1.0609x over previous
import functools

import jax
import jax.numpy as jnp
from jax.experimental import pallas as pl
from jax.experimental.pallas import tpu as pltpu


_PAR = pltpu.CompilerParams(dimension_semantics=("parallel",))
_BF = jnp.bfloat16
_F32 = jnp.float32


def _dot(a, b):
    return jnp.dot(a, b, preferred_element_type=_F32)


def _row(r, c):
    return pl.BlockSpec((r, c), lambda i: (i, 0))


def _full(a):
    return pl.BlockSpec(a.shape, lambda i: (0, 0))


def _sage_pair_body(asp_ref, aft_ref, f_ref, ws_ref, wn_ref,
                    esp_ref, eft_ref, absp_ref, *, r):
    i = pl.program_id(0)
    f = f_ref[...]
    wn = wn_ref[...]
    s = _dot(f_ref[pl.ds(i * r, r), :], ws_ref[...])
    asp = asp_ref[...]
    absp_ref[...] = asp.astype(_BF)
    nsp = _dot(asp, f)
    esp_ref[...] = jnp.maximum(_dot(nsp, wn) + s, 0.0)
    nft = _dot(aft_ref[...], f)
    eft_ref[...] = jnp.maximum(_dot(nft, wn) + s, 0.0)


def _sage_pair(asp, aft, feat, ws, wn, block_rows=200):
    n, k = asp.shape
    d = feat.shape[1]
    o = wn.shape[1]
    r = block_rows if n % block_rows == 0 else n
    return pl.pallas_call(
        functools.partial(_sage_pair_body, r=r), grid=(n // r,),
        in_specs=[_row(r, k), _row(r, k), _full(feat), _full(ws),
                  _full(wn)],
        out_specs=[_row(r, o), _row(r, o), _row(r, k)],
        out_shape=[jax.ShapeDtypeStruct((n, o), _F32),
                   jax.ShapeDtypeStruct((n, o), _F32),
                   jax.ShapeDtypeStruct((n, k), _BF)],
        compiler_params=_PAR,
    )(asp, aft, feat, ws, wn)


def _attn_pair(e1, e2, w, u):
    s1 = _dot(jnp.tanh(_dot(e1, w)), u)
    s2 = _dot(jnp.tanh(_dot(e2, w)), u)
    m = jnp.maximum(s1, s2)
    x1 = jnp.exp(s1 - m)
    x2 = jnp.exp(s2 - m)
    den = x1 + x2
    a1 = x1 / den
    a2 = x2 / den
    lat = (e1.astype(_BF).astype(_F32) * a1.astype(_BF).astype(_F32)
           + e2.astype(_BF).astype(_F32) * a2.astype(_BF).astype(_F32))
    return lat, a1, a2


def _sage_attn_body(asp_ref, aft_ref, f_ref, ws_ref, wn_ref,
                    esp1_ref, eft1_ref,
                    w1_ref, u1_ref, w2_ref, u2_ref, wc_ref, uc_ref,
                    lat1_ref, lat2_ref, comb_ref, ap_ref,
                    cu1_ref, cu2_ref, absp_ref, *, r):
    i = pl.program_id(0)
    f = f_ref[...]
    wn = wn_ref[...]
    s = _dot(f_ref[pl.ds(i * r, r), :], ws_ref[...])
    asp = asp_ref[...]
    absp_ref[...] = asp.astype(_BF)
    esp2 = jnp.maximum(_dot(_dot(asp, f), wn) + s, 0.0)
    eft2 = jnp.maximum(_dot(_dot(aft_ref[...], f), wn) + s, 0.0)
    lat1, a10, a11 = _attn_pair(esp1_ref[...], eft1_ref[...],
                                w1_ref[...], u1_ref[...])
    lat2, a20, a21 = _attn_pair(esp2, eft2, w2_ref[...], u2_ref[...])
    comb, ac0, ac1 = _attn_pair(lat1, lat2, wc_ref[...], uc_ref[...])
    lat1_ref[...] = lat1
    lat2_ref[...] = lat2
    comb_ref[...] = comb
    ap_ref[...] = jnp.concatenate(
        [a10, a11, a20, a21, ac0, ac1,
         jnp.zeros((r, 122), _F32)], axis=1)
    cu1_ref[...] = jnp.concatenate([comb, lat2], axis=1).astype(_BF)
    cu2_ref[...] = jnp.concatenate([comb, lat1], axis=1).astype(_BF)


def _sage_attn(asp, aft, feat, ws, wn, esp1, eft1,
               w1, u1, w2, u2, wc, uc, block_rows=200):
    n, k = asp.shape
    o = wn.shape[1]
    r = block_rows if n % block_rows == 0 else n
    return pl.pallas_call(
        functools.partial(_sage_attn_body, r=r), grid=(n // r,),
        in_specs=[_row(r, k), _row(r, k), _full(feat), _full(ws),
                  _full(wn), _row(r, o), _row(r, o),
                  _full(w1), _full(u1), _full(w2), _full(u2),
                  _full(wc), _full(uc)],
        out_specs=[_row(r, o), _row(r, o), _row(r, o), _row(r, 128),
                   _row(r, 2 * o), _row(r, 2 * o), _row(r, k)],
        out_shape=[jax.ShapeDtypeStruct((n, o), _F32),
                   jax.ShapeDtypeStruct((n, o), _F32),
                   jax.ShapeDtypeStruct((n, o), _F32),
                   jax.ShapeDtypeStruct((n, 128), _F32),
                   jax.ShapeDtypeStruct((n, 2 * o), _BF),
                   jax.ShapeDtypeStruct((n, 2 * o), _BF),
                   jax.ShapeDtypeStruct((n, k), _BF)],
        compiler_params=_PAR,
    )(asp, aft, feat, ws, wn, esp1, eft1,
      w1, u1, w2, u2, wc, uc)


def _spmm_epi_body(a_ref, cu_ref, comb_ref, latb_ref,
                   wsd_ref, wnd_ref, ws_ref, wn_ref,
                   rec_ref, sa_ref, ha_ref):
    o = comb_ref.shape[1]
    v = _dot(a_ref[...], cu_ref[...])
    wsd = wsd_ref[...]
    wnd = wnd_ref[...]
    rec_ref[...] = jnp.maximum(
        _dot(comb_ref[...], wsd) + _dot(v[:, :o], wnd), 0.0)
    ain = jnp.maximum(
        _dot(latb_ref[...], wsd) + _dot(v[:, o:], wnd), 0.0)
    sa_ref[...] = _dot(ain, ws_ref[...])
    ha_ref[...] = _dot(ain, wn_ref[...]).astype(_BF)


def _spmm_epi(adj, cu, comb, latb, wsd, wnd, ws, wn, block_rows=400):
    n, k = adj.shape
    o = comb.shape[1]
    dd = wsd.shape[1]
    oo = ws.shape[1]
    r = block_rows if n % block_rows == 0 else n
    return pl.pallas_call(
        _spmm_epi_body, grid=(n // r,),
        in_specs=[_row(r, k), _full(cu), _row(r, o), _row(r, o),
                  _full(wsd), _full(wnd), _full(ws), _full(wn)],
        out_specs=[_row(r, dd), _row(r, oo), _row(r, oo)],
        out_shape=[jax.ShapeDtypeStruct((n, dd), _F32),
                   jax.ShapeDtypeStruct((n, oo), _F32),
                   jax.ShapeDtypeStruct((n, oo), _BF)],
        compiler_params=_PAR,
    )(adj, cu, comb, latb, wsd, wnd, ws, wn)


def _dual_sage_body(a2_ref, a1_ref, h1_ref, h2_ref, s1_ref, s2_ref,
                    o1_ref, o2_ref):
    o1_ref[...] = jnp.maximum(
        _dot(a2_ref[...], h1_ref[...]) + s1_ref[...], 0.0)
    o2_ref[...] = jnp.maximum(
        _dot(a1_ref[...], h2_ref[...]) + s2_ref[...], 0.0)


def _dual_sage(a2, a1, h1, h2, s1, s2, block_rows=400):
    n, k = a2.shape
    o = h1.shape[1]
    r = block_rows if n % block_rows == 0 else n
    return pl.pallas_call(
        _dual_sage_body, grid=(n // r,),
        in_specs=[_row(r, k), _row(r, k), _full(h1), _full(h2),
                  _row(r, o), _row(r, o)],
        out_specs=[_row(r, o), _row(r, o)],
        out_shape=[jax.ShapeDtypeStruct((n, o), _F32),
                   jax.ShapeDtypeStruct((n, o), _F32)],
        compiler_params=_PAR,
    )(a2, a1, h1, h2, s1, s2)


def kernel(features_omics1, features_omics2, adj_spatial_omics1,
           adj_feature_omics1, adj_spatial_omics2, adj_feature_omics2,
           Ws1, Wn1, Wsd1, Wnd1, Ws2, Wn2, Wsd2, Wnd2,
           w1, u1, w2, u2, wc, uc):
    e_sp1, e_ft1, asp1_bf = _sage_pair(
        adj_spatial_omics1, adj_feature_omics1, features_omics1, Ws1, Wn1)
    lat1, lat2, comb, ap, cu1, cu2, asp2_bf = _sage_attn(
        adj_spatial_omics2, adj_feature_omics2, features_omics2, Ws2, Wn2,
        e_sp1, e_ft1, w1, u1, w2, u2, wc, uc)
    alpha1 = ap[:, 0:2]
    alpha2 = ap[:, 2:4]
    alpha12 = ap[:, 4:6]

    rec1, sa2, ha2 = _spmm_epi(asp1_bf, cu1, comb, lat2,
                               Wsd1, Wnd1, Ws1, Wn1)
    rec2, sa1, ha1 = _spmm_epi(asp2_bf, cu2, comb, lat1,
                               Wsd2, Wnd2, Ws2, Wn2)

    across1, across2 = _dual_sage(asp2_bf, asp1_bf,
                                  ha1, ha2, sa1, sa2)

    return (lat1, lat2, comb, rec1, rec2, across1, across2,
            alpha1, alpha2, alpha12)
